# baseline (speedup 1.0000x reference)
import functools

import jax
import jax.numpy as jnp
from jax import lax
from jax.experimental import pallas as pl
from jax.experimental.pallas import tpu as pltpu
from jax.experimental.pallas import tpu_sc as plsc

_NW = 32
_CHUNK = 128
_D = 32
_G = 4


@functools.partial(jax.jit, static_argnums=(4,))
def _run(idx1, idx2, w1, w2, n_chunks_per_w):
    total_chunks = _NW * n_chunks_per_w
    n_groups = n_chunks_per_w // _G
    mesh = plsc.VectorSubcoreMesh(core_axis_name="c", subcore_axis_name="s")

    @functools.partial(
        pl.kernel,
        mesh=mesh,
        out_type=jax.ShapeDtypeStruct((total_chunks, _CHUNK, _D), jnp.float32),
        compiler_params=pltpu.CompilerParams(use_tc_tiling_on_sc=False),
        scratch_types=[
            pltpu.VMEM((n_chunks_per_w, _CHUNK), jnp.int32),
            pltpu.VMEM((n_chunks_per_w, _CHUNK), jnp.int32),
            pltpu.VMEM((_G, _CHUNK, 128), jnp.float32),
            pltpu.SemaphoreType.DMA((_G,)),
        ],
    )
    def k(idx1_hbm, idx2_hbm, w1_hbm, w2_hbm, out_hbm, i1_v, i2_v, buf, sems):
        cid = lax.axis_index("c")
        sid = lax.axis_index("s")
        wid = sid * 2 + cid
        base = wid * n_chunks_per_w
        pltpu.sync_copy(idx1_hbm.at[wid], i1_v)
        pltpu.sync_copy(idx2_hbm.at[wid], i2_v)

        for j in range(_G):
            pltpu.async_copy(w1_hbm.at[i1_v.at[j]], buf.at[j], sems.at[j])

        def group_body(g, carry):
            for j in range(_G):
                c = g * _G + j
                pltpu.make_async_copy(
                    w1_hbm.at[i1_v.at[c]], buf.at[j], sems.at[j]).wait()
                pltpu.async_copy(
                    w2_hbm.at[i2_v.at[c]], buf.at[j], sems.at[j], add=True)
            for j in range(_G):
                c = g * _G + j
                pltpu.make_async_copy(
                    w2_hbm.at[i2_v.at[c]], buf.at[j], sems.at[j]).wait()
                pltpu.async_copy(buf.at[j, :, pl.ds(0, _D)], out_hbm.at[base + c], sems.at[j])
            for j in range(_G):
                c = g * _G + j
                pltpu.make_async_copy(
                    buf.at[j, :, pl.ds(0, _D)], out_hbm.at[base + c], sems.at[j]).wait()

                @pl.when(g + 1 < n_groups)
                def _():
                    c2 = (g + 1) * _G + j
                    pltpu.async_copy(
                        w1_hbm.at[i1_v.at[c2]], buf.at[j], sems.at[j])
            return carry

        lax.fori_loop(0, n_groups, group_body, 0)

    return k(idx1, idx2, w1, w2)


def kernel(input, another_input, W1, W2):
    B, H = input.shape
    total = B * H
    n_chunks_per_w = total // (_NW * _CHUNK)
    idx1 = input.reshape(_NW, n_chunks_per_w, _CHUNK).astype(jnp.int32)
    idx2 = another_input.reshape(_NW, n_chunks_per_w, _CHUNK).astype(jnp.int32)
    w1p = jnp.pad(W1, ((0, 0), (0, 128 - _D)))
    w2p = jnp.pad(W2, ((0, 0), (0, 128 - _D)))
    out = _run(idx1, idx2, w1p, w2p, n_chunks_per_w)
    return out.reshape(B, H, _D)


# speedup vs baseline: 1.3919x; 1.3919x over previous
import functools

import jax
import jax.numpy as jnp
from jax import lax
from jax.experimental import pallas as pl
from jax.experimental.pallas import tpu as pltpu
from jax.experimental.pallas import tpu_sc as plsc

_NW = 32
_CHUNK = 128
_D = 32
_G = 8
_H = 50
_B = 16384


@functools.partial(jax.jit, static_argnums=(4,))
def _run(idx1, idx2, w1, w2, n_chunks_per_w):
    mesh = plsc.VectorSubcoreMesh(core_axis_name="c", subcore_axis_name="s")

    @functools.partial(
        pl.kernel,
        mesh=mesh,
        out_type=jax.ShapeDtypeStruct((_H, _D // 8, _B // 128, 8, 128), jnp.float32),
        compiler_params=pltpu.CompilerParams(use_tc_tiling_on_sc=False, needs_layout_passes=False),
        scratch_types=[
            pltpu.VMEM((n_chunks_per_w, _CHUNK), jnp.int32),
            pltpu.VMEM((n_chunks_per_w, _CHUNK), jnp.int32),
            pltpu.VMEM((_G, _CHUNK, _D), jnp.float32),
            pltpu.VMEM((_G, _D // 8, 8, _CHUNK), jnp.float32),
            pltpu.SemaphoreType.DMA((_G,)),
        ],
    )
    def k(idx1_hbm, idx2_hbm, w1_hbm, w2_hbm, out_hbm,
          i1_v, i2_v, buf, tbuf, sems):
        cid = lax.axis_index("c")
        sid = lax.axis_index("s")
        wid = sid * 2 + cid
        base = wid * n_chunks_per_w
        pltpu.sync_copy(idx1_hbm.at[wid], i1_v)
        pltpu.sync_copy(idx2_hbm.at[wid], i2_v)

        lanes = lax.iota(jnp.int32, 16)

        def transpose_chunk(j):
            def body(t, carry):
                td = t // 8
                dr = t - td * 8
                for brblk in range(_CHUNK // 16):
                    rows = lanes + (brblk * 16)
                    vals = plsc.load_gather(
                        buf.at[j], [rows, jnp.full((16,), 0, jnp.int32) + t])
                    tbuf[j, td, dr, pl.ds(brblk * 16, 16)] = vals
                return carry
            lax.fori_loop(0, _D, body, 0)

        def fire_stores(j, h, tb):
            for td in range(_D // 8):
                pltpu.async_copy(
                    tbuf.at[j, td], out_hbm.at[h, td, tb], sems.at[j])

        def wait_stores(j, h, tb):
            for td in range(_D // 8):
                pltpu.make_async_copy(
                    tbuf.at[j, td], out_hbm.at[h, td, tb], sems.at[j]).wait()

        for j in range(_G):
            pltpu.async_copy(w1_hbm.at[i1_v.at[j]], buf.at[j], sems.at[j])

        def group_body(g, carry):
            for j in range(_G):
                c = g * _G + j
                pltpu.make_async_copy(
                    w1_hbm.at[i1_v.at[c]], buf.at[j], sems.at[j]).wait()
                pltpu.async_copy(
                    w2_hbm.at[i2_v.at[c]], buf.at[j], sems.at[j], add=True)
            for j in range(_G):
                c = g * _G + j
                gc = base + c
                h = gc // (_B // _CHUNK)
                tb = gc - h * (_B // _CHUNK)
                pltpu.make_async_copy(
                    w2_hbm.at[i2_v.at[c]], buf.at[j], sems.at[j]).wait()
                transpose_chunk(j)
                fire_stores(j, h, tb)
            for j in range(_G):
                c = g * _G + j
                gc = base + c
                h = gc // (_B // _CHUNK)
                tb = gc - h * (_B // _CHUNK)
                wait_stores(j, h, tb)

                @pl.when(g + 1 < n_chunks_per_w // _G)
                def _():
                    c2 = (g + 1) * _G + j
                    pltpu.async_copy(
                        w1_hbm.at[i1_v.at[c2]], buf.at[j], sems.at[j])
            return carry

        lax.fori_loop(0, n_chunks_per_w // _G, group_body, 0)

    return k(idx1, idx2, w1, w2)


def kernel(input, another_input, W1, W2):
    B, H = input.shape
    total = B * H
    n_chunks_per_w = total // (_NW * _CHUNK)
    idx1 = input.T.reshape(_NW, n_chunks_per_w, _CHUNK).astype(jnp.int32)
    idx2 = another_input.T.reshape(_NW, n_chunks_per_w, _CHUNK).astype(jnp.int32)
    out_p = _run(idx1, idx2, W1, W2, n_chunks_per_w)
    return out_p.transpose(2, 4, 0, 1, 3).reshape(B, H, _D)
